# baseline (speedup 1.0000x reference)
import functools

import jax
import jax.numpy as jnp
from jax import lax
from jax.experimental import pallas as pl
from jax.experimental.pallas import tpu as pltpu
from jax.experimental.pallas import tpu_sc as plsc

VOCAB = 49408
EMBED = 256
SEQ = 77
SEQ_PAD = 80
BATCH = 4096
NUM_WORKERS = 32
SEQ_PER_W = BATCH // NUM_WORKERS
LANES = 16
VREGS_PER_ROW = EMBED // LANES


@jax.jit
def _sc_embed(ids3, weight, position_embedding):
    mesh = plsc.VectorSubcoreMesh(core_axis_name="c", subcore_axis_name="s")

    @functools.partial(
        pl.kernel,
        out_type=jax.ShapeDtypeStruct((BATCH, SEQ, EMBED), jnp.float32),
        mesh=mesh,
        scratch_types=[
            pltpu.VMEM((SEQ_PER_W, SEQ_PAD), jnp.int32),
            pltpu.VMEM((SEQ, EMBED), jnp.float32),
            pltpu.VMEM((SEQ_PAD, EMBED), jnp.float32),
            pltpu.VMEM((SEQ_PAD, EMBED), jnp.float32),
            pltpu.VMEM((SEQ, EMBED), jnp.float32),
            pltpu.SemaphoreType.DMA,
            pltpu.SemaphoreType.DMA,
        ],
    )
    def k(ids_hbm, w_hbm, pos_hbm, out_hbm, idx_v, pos_v,
          rows_a, rows_b, fin_v, sem_a, sem_b):
        wid = lax.axis_index("s") * 2 + lax.axis_index("c")
        base = wid * SEQ_PER_W
        pltpu.sync_copy(ids_hbm.at[wid], idx_v)
        pltpu.sync_copy(pos_hbm, pos_v)

        def gather(s, buf, sem):
            return pltpu.async_copy(w_hbm.at[idx_v.at[s]], buf, sem)

        def add_and_flush(s, buf):
            @plsc.parallel_loop(0, SEQ, unroll=7)
            def add_body(r):
                for j in range(VREGS_PER_ROW):
                    sl = pl.ds(j * LANES, LANES)
                    fin_v[r, sl] = buf[r, sl] + pos_v[r, sl]

            pltpu.sync_copy(fin_v, out_hbm.at[base + s])

        gather(0, rows_a, sem_a)

        def pair_body(i, carry):
            s0 = 2 * i
            pltpu.make_async_copy(w_hbm.at[idx_v.at[s0]], rows_a, sem_a).wait()
            gather(s0 + 1, rows_b, sem_b)
            add_and_flush(s0, rows_a)
            pltpu.make_async_copy(w_hbm.at[idx_v.at[s0 + 1]], rows_b, sem_b).wait()
            gather(s0 + 2, rows_a, sem_a)
            add_and_flush(s0 + 1, rows_b)
            return carry

        lax.fori_loop(0, SEQ_PER_W // 2 - 1, pair_body, 0)
        s0 = SEQ_PER_W - 2
        pltpu.make_async_copy(w_hbm.at[idx_v.at[s0]], rows_a, sem_a).wait()
        gather(s0 + 1, rows_b, sem_b)
        add_and_flush(s0, rows_a)
        pltpu.make_async_copy(w_hbm.at[idx_v.at[s0 + 1]], rows_b, sem_b).wait()
        add_and_flush(s0 + 1, rows_b)

    return k(ids3, weight, position_embedding)


def kernel(input_ids, weight, position_embedding):
    ids = jnp.asarray(input_ids, jnp.int32)
    ids3 = jnp.pad(ids, ((0, 0), (0, SEQ_PAD - SEQ))).reshape(
        NUM_WORKERS, SEQ_PER_W, SEQ_PAD)
    return _sc_embed(ids3, weight, position_embedding)


# speedup vs baseline: 2.2080x; 2.2080x over previous
import functools

import jax
import jax.numpy as jnp
from jax import lax
from jax.experimental import pallas as pl
from jax.experimental.pallas import tpu as pltpu
from jax.experimental.pallas import tpu_sc as plsc

VOCAB = 49408
EMBED = 256
SEQ = 77
BATCH = 4096
NUM_WORKERS = 32
IDS_PER_W = BATCH * SEQ // NUM_WORKERS
SEQ_PER_W = BATCH // NUM_WORKERS
CHUNK = 88
NCH = IDS_PER_W // CHUNK
RING = 3 * CHUNK
LANES = 16
VREGS_PER_ROW = EMBED // LANES


@jax.jit
def _sc_embed(ids3, weight, position_embedding):
    mesh = plsc.VectorSubcoreMesh(core_axis_name="c", subcore_axis_name="s")

    @functools.partial(
        pl.kernel,
        out_type=jax.ShapeDtypeStruct((BATCH, SEQ, EMBED), jnp.float32),
        mesh=mesh,
        scratch_types=[
            pltpu.VMEM((NCH, CHUNK), jnp.int32),
            pltpu.VMEM((SEQ, EMBED), jnp.float32),
            pltpu.VMEM((RING, EMBED), jnp.float32),
            pltpu.VMEM((SEQ, EMBED), jnp.float32),
            pltpu.SemaphoreType.DMA,
            pltpu.SemaphoreType.DMA,
            pltpu.SemaphoreType.DMA,
        ],
    )
    def k(ids_hbm, w_hbm, pos_hbm, out_hbm, idx_v, pos_v, ring_v, fin_v,
          sem0, sem1, sem2):
        sems = (sem0, sem1, sem2)
        wid = lax.axis_index("s") * 2 + lax.axis_index("c")
        sbase = wid * SEQ_PER_W
        pltpu.sync_copy(ids_hbm.at[wid], idx_v)
        pltpu.sync_copy(pos_hbm, pos_v)

        def gather(c, slot):
            return pltpu.async_copy(
                w_hbm.at[idx_v.at[c]],
                ring_v.at[pl.ds(slot * CHUNK, CHUNK)], sems[slot])

        def wait_gather(c, slot):
            pltpu.make_async_copy(
                w_hbm.at[idx_v.at[c]],
                ring_v.at[pl.ds(slot * CHUNK, CHUNK)], sems[slot]).wait()

        def emit_seq(s):
            q = lax.rem(s * SEQ, RING)

            @plsc.parallel_loop(0, SEQ, unroll=7)
            def add_body(r):
                rr = lax.rem(q + r, RING)
                for j in range(VREGS_PER_ROW):
                    sl = pl.ds(j * LANES, LANES)
                    fin_v[r, sl] = ring_v[rr, sl] + pos_v[r, sl]

            pltpu.sync_copy(fin_v, out_hbm.at[sbase + s])

        def drain_seqs(c, s):
            for _ in range(2):
                ready = (s < SEQ_PER_W) & (s * SEQ + SEQ - 1 < (c + 1) * CHUNK)

                @pl.when(ready)
                def _():
                    emit_seq(s)

                s = s + ready.astype(jnp.int32)
            return s

        def step(c, slot, s, issue):
            wait_gather(c, slot)
            s = drain_seqs(c, s)
            if issue:
                gather(c + 2, (slot + 2) % 3)
            return s

        gather(0, 0)
        gather(1, 1)

        def group_body(i, s):
            c = 3 * i
            s = step(c, 0, s, True)
            s = step(c + 1, 1, s, True)
            s = step(c + 2, 2, s, True)
            return s

        s = lax.fori_loop(0, NCH // 3 - 1, group_body, 0)
        s = step(NCH - 4, 0, s, True)
        s = step(NCH - 3, 1, s, True)
        s = step(NCH - 2, 2, s, False)
        s = step(NCH - 1, 0, s, False)

    return k(ids3, weight, position_embedding)


def kernel(input_ids, weight, position_embedding):
    ids3 = jnp.asarray(input_ids, jnp.int32).reshape(NUM_WORKERS, NCH, CHUNK)
    return _sc_embed(ids3, weight, position_embedding)


# speedup vs baseline: 2.3137x; 1.0479x over previous
import functools

import jax
import jax.numpy as jnp
from jax import lax
from jax.experimental import pallas as pl
from jax.experimental.pallas import tpu as pltpu
from jax.experimental.pallas import tpu_sc as plsc

VOCAB = 49408
EMBED = 256
SEQ = 77
BATCH = 4096
NUM_WORKERS = 32
IDS_PER_W = BATCH * SEQ // NUM_WORKERS
SEQ_PER_W = BATCH // NUM_WORKERS
CHUNK = 88
NCH = IDS_PER_W // CHUNK
RING = 3 * CHUNK
LANES = 16
GROUPS_PER_ROW = EMBED // (2 * LANES)


@jax.jit
def _sc_embed(ids2, weight, position_embedding):
    mesh = plsc.VectorSubcoreMesh(core_axis_name="c", subcore_axis_name="s")

    @functools.partial(
        pl.kernel,
        out_type=jax.ShapeDtypeStruct((BATCH, SEQ, EMBED), jnp.float32),
        mesh=mesh,
        scratch_types=[
            pltpu.VMEM((IDS_PER_W,), jnp.int32),
            pltpu.VMEM((SEQ, EMBED), jnp.float32),
            pltpu.VMEM((RING, EMBED), jnp.float32),
            pltpu.VMEM((SEQ, EMBED), jnp.float32),
            pltpu.SemaphoreType.DMA,
            pltpu.SemaphoreType.DMA,
            pltpu.SemaphoreType.DMA,
            pltpu.SemaphoreType.DMA,
        ],
    )
    def k(ids_hbm, w_hbm, pos_hbm, out_hbm, idx_v, pos_v, ring_v, fin_v,
          sem0, sem1, sem2, fsem):
        sems = (sem0, sem1, sem2)
        wid = lax.axis_index("s") * 2 + lax.axis_index("c")
        sbase = wid * SEQ_PER_W
        pltpu.sync_copy(ids_hbm.at[wid], idx_v)
        pltpu.sync_copy(pos_hbm, pos_v)

        def gather(c, slot):
            return pltpu.async_copy(
                w_hbm.at[idx_v.at[pl.ds(c * CHUNK, CHUNK)]],
                ring_v.at[pl.ds(slot * CHUNK, CHUNK)], sems[slot])

        def wait_gather(c, slot):
            pltpu.make_async_copy(
                w_hbm.at[idx_v.at[pl.ds(c * CHUNK, CHUNK)]],
                ring_v.at[pl.ds(slot * CHUNK, CHUNK)], sems[slot]).wait()

        def emit_seq(s):
            @pl.when(s >= 1)
            def _():
                pltpu.make_async_copy(
                    fin_v, out_hbm.at[sbase + s - 1], fsem).wait()

            q = lax.rem(s * SEQ, RING)

            @plsc.parallel_loop(0, SEQ, unroll=7)
            def add_body(r):
                rr = lax.rem(q + r, RING)
                for j in range(EMBED // LANES):
                    sl = pl.ds(j * LANES, LANES)
                    fin_v[r, sl] = ring_v[rr, sl] + pos_v[r, sl]

            pltpu.async_copy(fin_v, out_hbm.at[sbase + s], fsem)

        def drain_seqs(c, s):
            for _ in range(2):
                ready = (s < SEQ_PER_W) & (s * SEQ + SEQ - 1 < (c + 1) * CHUNK)

                @pl.when(ready)
                def _():
                    emit_seq(s)

                s = s + ready.astype(jnp.int32)
            return s

        def step(c, slot, s, issue):
            wait_gather(c, slot)
            s = drain_seqs(c, s)
            if issue:
                gather(c + 2, (slot + 2) % 3)
            return s

        gather(0, 0)
        gather(1, 1)

        def group_body(i, s):
            c = 3 * i
            s = step(c, 0, s, True)
            s = step(c + 1, 1, s, True)
            s = step(c + 2, 2, s, True)
            return s

        s = lax.fori_loop(0, NCH // 3 - 1, group_body, 0)
        s = step(NCH - 4, 0, s, True)
        s = step(NCH - 3, 1, s, True)
        s = step(NCH - 2, 2, s, False)
        s = step(NCH - 1, 0, s, False)
        pltpu.make_async_copy(
            fin_v, out_hbm.at[sbase + SEQ_PER_W - 1], fsem).wait()

    return k(ids2, weight, position_embedding)


def kernel(input_ids, weight, position_embedding):
    ids2 = jnp.asarray(input_ids, jnp.int32).reshape(NUM_WORKERS, IDS_PER_W)
    return _sc_embed(ids2, weight, position_embedding)


# speedup vs baseline: 2.3294x; 1.0068x over previous
import functools

import jax
import jax.numpy as jnp
from jax import lax
from jax.experimental import pallas as pl
from jax.experimental.pallas import tpu as pltpu
from jax.experimental.pallas import tpu_sc as plsc

VOCAB = 49408
EMBED = 256
SEQ = 77
BATCH = 4096
NUM_WORKERS = 32
IDS_PER_W = BATCH * SEQ // NUM_WORKERS
SEQ_PER_W = BATCH // NUM_WORKERS
CHUNK = 88
NCH = IDS_PER_W // CHUNK
RING = 3 * CHUNK
LANES = 16
GROUPS_PER_ROW = EMBED // (2 * LANES)


@jax.jit
def _sc_embed(ids2, weight, position_embedding):
    mesh = plsc.VectorSubcoreMesh(core_axis_name="c", subcore_axis_name="s")

    @functools.partial(
        pl.kernel,
        out_type=jax.ShapeDtypeStruct((BATCH, SEQ, EMBED), jnp.float32),
        mesh=mesh,
        scratch_types=[
            pltpu.VMEM((IDS_PER_W,), jnp.int32),
            pltpu.VMEM((SEQ, EMBED), jnp.float32),
            pltpu.VMEM((RING, EMBED), jnp.float32),
            pltpu.VMEM((SEQ, EMBED), jnp.float32),
            pltpu.SemaphoreType.DMA,
            pltpu.SemaphoreType.DMA,
            pltpu.SemaphoreType.DMA,
            pltpu.SemaphoreType.DMA,
        ],
    )
    def k(ids_hbm, w_hbm, pos_hbm, out_hbm, idx_v, pos_v, ring_v, fin_v,
          sem0, sem1, sem2, fsem):
        sems = (sem0, sem1, sem2)
        wid = lax.axis_index("s") * 2 + lax.axis_index("c")
        sbase = wid * SEQ_PER_W
        pltpu.sync_copy(ids_hbm.at[wid], idx_v)
        pltpu.sync_copy(pos_hbm, pos_v)

        def gather(c, slot):
            return pltpu.async_copy(
                w_hbm.at[idx_v.at[pl.ds(c * CHUNK, CHUNK)]],
                ring_v.at[pl.ds(slot * CHUNK, CHUNK)], sems[slot])

        def wait_gather(c, slot):
            pltpu.make_async_copy(
                w_hbm.at[idx_v.at[pl.ds(c * CHUNK, CHUNK)]],
                ring_v.at[pl.ds(slot * CHUNK, CHUNK)], sems[slot]).wait()

        def emit_seq(s):
            @pl.when(s >= 1)
            def _():
                pltpu.make_async_copy(
                    fin_v, out_hbm.at[sbase + s - 1], fsem).wait()

            q = lax.rem(s * SEQ, RING)

            @plsc.parallel_loop(0, SEQ, unroll=7)
            def add_body(r):
                rr = lax.rem(q + r, RING)
                for j in range(EMBED // LANES):
                    sl = pl.ds(j * LANES, LANES)
                    fin_v[r, sl] = ring_v[rr, sl] + pos_v[r, sl]

            pltpu.async_copy(fin_v, out_hbm.at[sbase + s], fsem)

        def emit_if_ready(c, s):
            ready = (s < SEQ_PER_W) & (s * SEQ + SEQ - 1 < (c + 1) * CHUNK)

            @pl.when(ready)
            def _():
                emit_seq(s)

            return s + ready.astype(jnp.int32)

        def step(c, slot, s, issue):
            wait_gather(c, slot)
            s = emit_if_ready(c, s)
            if issue:
                gather(c + 2, (slot + 2) % 3)
            s = emit_if_ready(c, s)
            return s

        gather(0, 0)
        gather(1, 1)

        def group_body(i, s):
            c = 3 * i
            s = step(c, 0, s, True)
            s = step(c + 1, 1, s, True)
            s = step(c + 2, 2, s, True)
            return s

        s = lax.fori_loop(0, NCH // 3 - 1, group_body, 0)
        s = step(NCH - 4, 0, s, True)
        s = step(NCH - 3, 1, s, True)
        s = step(NCH - 2, 2, s, False)
        s = step(NCH - 1, 0, s, False)
        pltpu.make_async_copy(
            fin_v, out_hbm.at[sbase + SEQ_PER_W - 1], fsem).wait()

    return k(ids2, weight, position_embedding)


def kernel(input_ids, weight, position_embedding):
    ids2 = jnp.asarray(input_ids, jnp.int32).reshape(NUM_WORKERS, IDS_PER_W)
    return _sc_embed(ids2, weight, position_embedding)
